# baseline (speedup 1.0000x reference)
import functools

import jax
import jax.numpy as jnp
from jax import lax
from jax.experimental import pallas as pl
from jax.experimental.pallas import tpu as pltpu
from jax.experimental.pallas import tpu_sc as plsc

_B = 1024
_E = 32
_V = 100000
_BN = 2048


@functools.lru_cache(maxsize=None)
def _make_sc_gather(num_cores: int, num_subcores: int):
    nw = num_cores * num_subcores
    b_per_w = _B // nw
    mesh = plsc.VectorSubcoreMesh(core_axis_name="c", subcore_axis_name="s")

    @functools.partial(
        pl.kernel,
        mesh=mesh,
        out_type=jax.ShapeDtypeStruct((_B, _E), jnp.float32),
        scratch_types=[
            pltpu.VMEM((b_per_w,), jnp.int32),
            pltpu.VMEM((b_per_w, _E), jnp.float32),
            pltpu.SemaphoreType.DMA,
        ],
        compiler_params=pltpu.CompilerParams(use_tc_tiling_on_sc=False),
    )
    def gather(table_hbm, idx_hbm, out_hbm, idx_v, rows_v, sem):
        wid = lax.axis_index("s") * num_cores + lax.axis_index("c")
        base = wid * b_per_w
        pltpu.sync_copy(idx_hbm.at[pl.ds(base, b_per_w)], idx_v)
        pltpu.async_copy(table_hbm.at[idx_v], rows_v, sem).wait()
        pltpu.sync_copy(rows_v, out_hbm.at[pl.ds(base, b_per_w)])

    return gather


def _mm_body(emb_ref, w_ref, b_ref, out_ref):
    out_ref[...] = (
        jnp.dot(emb_ref[...], w_ref[...], preferred_element_type=jnp.float32)
        + b_ref[...]
    )


def _tc_project(emb, dense_kernel, bias2d):
    grid = pl.cdiv(_V, _BN)
    return pl.pallas_call(
        _mm_body,
        out_shape=jax.ShapeDtypeStruct((_B, _V), jnp.float32),
        grid=(grid,),
        in_specs=[
            pl.BlockSpec((_B, _E), lambda i: (0, 0)),
            pl.BlockSpec((_E, _BN), lambda i: (0, i)),
            pl.BlockSpec((1, _BN), lambda i: (0, i)),
        ],
        out_specs=pl.BlockSpec((_B, _BN), lambda i: (0, i)),
        compiler_params=pltpu.CompilerParams(
            dimension_semantics=("arbitrary",),
        ),
    )(emb, dense_kernel, bias2d)


def kernel(function_type_ids, embedding_table, dense_kernel, dense_bias):
    info = plsc.get_sparse_core_info()
    ids = function_type_ids.astype(jnp.int32)
    emb = _make_sc_gather(info.num_cores, info.num_subcores)(
        embedding_table, ids
    )
    return _tc_project(emb, dense_kernel, dense_bias.reshape(1, _V))


# speedup vs baseline: 1.0043x; 1.0043x over previous
import functools

import jax
import jax.numpy as jnp
from jax import lax
from jax.experimental import pallas as pl
from jax.experimental.pallas import tpu as pltpu
from jax.experimental.pallas import tpu_sc as plsc

_B = 1024
_E = 32
_V = 100000
_BN = 2048


@functools.lru_cache(maxsize=None)
def _make_sc_gather(num_cores: int, num_subcores: int):
    nw = num_cores * num_subcores
    b_per_w = _B // nw
    mesh = plsc.VectorSubcoreMesh(core_axis_name="c", subcore_axis_name="s")

    @functools.partial(
        pl.kernel,
        mesh=mesh,
        out_type=jax.ShapeDtypeStruct((_B, _E), jnp.float32),
        scratch_types=[
            pltpu.VMEM((b_per_w,), jnp.int32),
            pltpu.VMEM((b_per_w, _E), jnp.float32),
            pltpu.SemaphoreType.DMA,
        ],
        compiler_params=pltpu.CompilerParams(use_tc_tiling_on_sc=False),
    )
    def gather(table_hbm, idx_hbm, out_hbm, idx_v, rows_v, sem):
        wid = lax.axis_index("s") * num_cores + lax.axis_index("c")
        base = wid * b_per_w
        pltpu.sync_copy(idx_hbm.at[pl.ds(base, b_per_w)], idx_v)
        pltpu.async_copy(table_hbm.at[idx_v], rows_v, sem).wait()
        pltpu.sync_copy(rows_v, out_hbm.at[pl.ds(base, b_per_w)])

    return gather


def _mm_body(emb_ref, w_ref, b_ref, out_ref):
    out_ref[...] = (
        jnp.dot(emb_ref[...], w_ref[...], preferred_element_type=jnp.float32)
        + b_ref[...]
    )


_BM = 16


def _tc_project(emb, dense_kernel, bias2d):
    grid = _B // _BM
    return pl.pallas_call(
        _mm_body,
        out_shape=jax.ShapeDtypeStruct((_B, _V), jnp.float32),
        grid=(grid,),
        in_specs=[
            pl.BlockSpec((_BM, _E), lambda i: (i, 0)),
            pl.BlockSpec((_E, _V), lambda i: (0, 0)),
            pl.BlockSpec((1, _V), lambda i: (0, 0)),
        ],
        out_specs=pl.BlockSpec((_BM, _V), lambda i: (i, 0)),
        compiler_params=pltpu.CompilerParams(
            dimension_semantics=("parallel",),
        ),
    )(emb, dense_kernel, bias2d)


def kernel(function_type_ids, embedding_table, dense_kernel, dense_bias):
    info = plsc.get_sparse_core_info()
    ids = function_type_ids.astype(jnp.int32)
    emb = _make_sc_gather(info.num_cores, info.num_subcores)(
        embedding_table, ids
    )
    return _tc_project(emb, dense_kernel, dense_bias.reshape(1, _V))
